# Initial kernel scaffold; baseline (speedup 1.0000x reference)
import jax
import jax.numpy as jnp
from jax.experimental import pallas as pl


def kernel(x, V_idx, E_idx, edge_ids, W_node, edge_factors):
    raise NotImplementedError("write your pallas kernel here")


# speedup vs baseline: 19.6181x; 19.6181x over previous
import functools

import jax
import jax.numpy as jnp
from jax import lax
from jax.experimental import pallas as pl
from jax.experimental.pallas import tpu as pltpu
from jax.experimental.pallas import tpu_sc as plsc

NC = 2
NS = 16
NW = NC * NS
L = 16

L1_BINS = 512
L2_BINS = 2048
L3_BINS = 1024

_MESH = plsc.VectorSubcoreMesh(core_axis_name="c", subcore_axis_name="s")


def _wid():
    return lax.axis_index("s") * NC + lax.axis_index("c")


def _proj_body(x_ref, w_ref, out_ref):
    out_ref[...] = lax.dot_general(
        x_ref[...], w_ref[...],
        dimension_numbers=(((1,), (1,)), ((), ())),
        preferred_element_type=jnp.float32)


def _node_proj(x, w_node):
    n = x.shape[0]
    r = w_node.shape[0]
    return pl.pallas_call(
        _proj_body,
        out_shape=jax.ShapeDtypeStruct((n, r), jnp.float32),
    )(x, w_node)


def _scores_body(nnz, blk, node_hbm, vidx_hbm, eidx_hbm, eidtab_hbm, ef_hbm,
                 scores_hbm, hist1_hbm,
                 eidtab_v, vv_v, ve_v, eids_v, node_b, ef_b, sc_b,
                 hist_v, merged_v, sem):
    chunk = nnz // NW
    nblk = chunk // blk
    grp = blk // L
    wid = _wid()
    base = wid * chunk
    iota = lax.iota(jnp.int32, L)
    ones_i = jnp.ones((L,), jnp.int32)

    pltpu.sync_copy(eidtab_hbm, eidtab_v)

    def zh(i, _):
        hist_v[i, :] = jnp.zeros((L,), jnp.int32)
        return 0
    lax.fori_loop(0, L1_BINS, zh, 0)

    def blk_body(j, _):
        off = base + j * blk
        pltpu.sync_copy(vidx_hbm.at[pl.ds(off, blk)], vv_v)
        pltpu.sync_copy(eidx_hbm.at[pl.ds(off, blk)], ve_v)

        def eg(i, _):
            idx = ve_v[pl.ds(i * L, L)]
            eids_v[pl.ds(i * L, L)] = plsc.load_gather(eidtab_v, [idx])
            return 0
        lax.fori_loop(0, grp, eg, 0)

        slices = [(c, min(128, blk - c)) for c in range(0, blk, 128)]
        for g in range(0, len(slices), 4):
            descs = []
            for (c, sz) in slices[g:g + 4]:
                descs.append(pltpu.async_copy(
                    node_hbm.at[vv_v.at[pl.ds(c, sz)]],
                    node_b.at[pl.ds(c, sz)], sem))
                descs.append(pltpu.async_copy(
                    ef_hbm.at[eids_v.at[pl.ds(c, sz)]],
                    ef_b.at[pl.ds(c, sz)], sem))
            for d in descs:
                d.wait()

        def grp_body(i, _):
            rows = i * L + iota
            acc = jnp.zeros((L,), jnp.float32)
            for cc in range(L):
                col = jnp.full((L,), cc, jnp.int32)
                nv = plsc.load_gather(node_b, [rows, col])
                ev = plsc.load_gather(ef_b, [rows, col])
                acc = acc + nv * ev
            sig = 1.0 / (1.0 + jnp.exp(-acc))
            sc_b[pl.ds(i * L, L)] = sig
            key = plsc.bitcast(sig, jnp.int32)
            bin1 = lax.shift_right_logical(key, 21)
            plsc.addupdate_scatter(hist_v, [bin1, iota], ones_i)
            return 0
        lax.fori_loop(0, grp, grp_body, 0)
        pltpu.sync_copy(sc_b, scores_hbm.at[pl.ds(off, blk)])
        return 0
    lax.fori_loop(0, nblk, blk_body, 0)

    def mg(bq, _):
        rows = bq * L + iota
        accm = jnp.zeros((L,), jnp.int32)
        for cc in range(L):
            col = jnp.full((L,), cc, jnp.int32)
            accm = accm + plsc.load_gather(hist_v, [rows, col])
        merged_v[pl.ds(bq * L, L)] = accm
        return 0
    lax.fori_loop(0, L1_BINS // L, mg, 0)
    pltpu.sync_copy(merged_v, hist1_hbm.at[wid])


def _scores_and_hist1(node_r, v_idx, e_idx, edge_ids, edge_factors, blk=2000):
    nnz = v_idx.shape[0]
    ne = edge_ids.shape[0]
    body = functools.partial(_scores_body, nnz, blk)
    f = pl.kernel(
        body,
        out_type=(jax.ShapeDtypeStruct((nnz,), jnp.float32),
                  jax.ShapeDtypeStruct((NW, L1_BINS), jnp.int32)),
        mesh=_MESH,
        compiler_params=pltpu.CompilerParams(needs_layout_passes=False,
                                             use_tc_tiling_on_sc=False),
        scratch_types=[
            pltpu.VMEM((ne,), jnp.int32),
            pltpu.VMEM((blk,), jnp.int32),
            pltpu.VMEM((blk,), jnp.int32),
            pltpu.VMEM((blk,), jnp.int32),
            pltpu.VMEM((blk, L), jnp.float32),
            pltpu.VMEM((blk, L), jnp.float32),
            pltpu.VMEM((blk,), jnp.float32),
            pltpu.VMEM((L1_BINS, L), jnp.int32),
            pltpu.VMEM((L1_BINS,), jnp.int32),
            pltpu.SemaphoreType.DMA,
        ],
    )
    return f(node_r, v_idx, e_idx, edge_ids, edge_factors)


def _hist_body(nnz, level, scores_hbm, params_hbm, out_hbm,
               sc_v, hist_v, merged_v, params_v):
    nbins = L2_BINS if level == 2 else L3_BINS
    chunk = nnz // NW
    grp = chunk // L
    wid = _wid()
    base = wid * chunk
    iota = lax.iota(jnp.int32, L)
    ones_i = jnp.ones((L,), jnp.int32)

    pltpu.sync_copy(params_hbm.at[0], params_v)
    head = params_v[pl.ds(0, L)]
    pref = head[0] if level == 2 else head[2]

    pltpu.sync_copy(scores_hbm.at[pl.ds(base, chunk)], sc_v)

    def zh(i, _):
        hist_v[i, :] = jnp.zeros((L,), jnp.int32)
        return 0
    lax.fori_loop(0, nbins, zh, 0)

    def grp_body(i, _):
        key = plsc.bitcast(sc_v[pl.ds(i * L, L)], jnp.int32)
        if level == 2:
            match = lax.shift_right_logical(key, 21) == pref
            b = lax.shift_right_logical(key, 10) & (L2_BINS - 1)
        else:
            match = lax.shift_right_logical(key, 10) == pref
            b = key & (L3_BINS - 1)
        plsc.addupdate_scatter(hist_v, [b, iota], ones_i, mask=match)
        return 0
    lax.fori_loop(0, grp, grp_body, 0)

    def mg(bq, _):
        rows = bq * L + iota
        accm = jnp.zeros((L,), jnp.int32)
        for cc in range(L):
            col = jnp.full((L,), cc, jnp.int32)
            accm = accm + plsc.load_gather(hist_v, [rows, col])
        merged_v[pl.ds(bq * L, L)] = accm
        return 0
    lax.fori_loop(0, nbins // L, mg, 0)
    pltpu.sync_copy(merged_v, out_hbm.at[wid])


def _hist_level(scores, params, level):
    nnz = scores.shape[0]
    nbins = L2_BINS if level == 2 else L3_BINS
    body = functools.partial(_hist_body, nnz, level)
    f = pl.kernel(
        body,
        out_type=jax.ShapeDtypeStruct((NW, nbins), jnp.int32),
        mesh=_MESH,
        compiler_params=pltpu.CompilerParams(needs_layout_passes=False,
                                             use_tc_tiling_on_sc=False),
        scratch_types=[
            pltpu.VMEM((nnz // NW,), jnp.float32),
            pltpu.VMEM((nbins, L), jnp.int32),
            pltpu.VMEM((nbins,), jnp.int32),
            pltpu.VMEM((128,), jnp.int32),
        ],
    )
    return f(scores, params)


def _count_ge(h2):
    rows = h2.shape[0]
    lanes = lax.broadcasted_iota(jnp.int32, (rows, 128), 1)
    suffix_within = h2
    for d in (1, 2, 4, 8, 16, 32, 64):
        rolled = pltpu.roll(suffix_within, 128 - d, 1)
        suffix_within = suffix_within + jnp.where(lanes < 128 - d, rolled, 0)
    rowsum = jnp.sum(h2, axis=1, keepdims=True)
    ri = lax.broadcasted_iota(jnp.int32, (rows, rows), 0)
    rj = lax.broadcasted_iota(jnp.int32, (rows, rows), 1)
    rs_b = jnp.broadcast_to(rowsum.reshape(1, rows), (rows, rows))
    below = jnp.sum(jnp.where(rj > ri, rs_b, 0), axis=1,
                    keepdims=True)
    return suffix_within + below


def _pick_core(hist_ref, kk):
    nbins = hist_ref.shape[1]
    rows = nbins // 128
    h2 = jnp.sum(hist_ref[...], axis=0).reshape(rows, 128)
    cg = _count_ge(h2)
    flat = (lax.broadcasted_iota(jnp.int32, (rows, 128), 0) * 128 +
            lax.broadcasted_iota(jnp.int32, (rows, 128), 1))
    mask = cg >= kk
    bsel = jnp.max(jnp.where(mask, flat, -1))
    hb = jnp.sum(jnp.where(flat == bsel, h2, 0))
    cgb = jnp.sum(jnp.where(flat == bsel, cg, 0))
    knext = kk - (cgb - hb)
    return bsel, knext


def _lane_vec(pairs, default=0):
    lanes = lax.broadcasted_iota(jnp.int32, (1, 128), 1)
    out = jnp.full((1, 128), default, jnp.int32)
    for lane, val in pairs:
        out = jnp.where(lanes == lane, val, out)
    return out


def _pick1_body(k, hist_ref, out_ref):
    b1, k2 = _pick_core(hist_ref, jnp.int32(k))
    out_ref[...] = _lane_vec([(0, b1), (1, k2)])


def _pick2_body(hist_ref, p1_ref, out_ref):
    kk = p1_ref[0, 1]
    b1 = p1_ref[0, 0]
    b2, k3 = _pick_core(hist_ref, kk)
    pref = lax.shift_left(b1, 11) | b2
    out_ref[...] = _lane_vec([(0, b2), (1, k3), (2, pref)])


def _pick3_body(hist_ref, p2_ref, out_ref):
    kk = p2_ref[0, 1]
    pref = p2_ref[0, 2]
    b3, rem = _pick_core(hist_ref, kk)
    t = lax.shift_left(pref, 10) | b3
    hi = hist_ref[...]
    bins = lax.broadcasted_iota(jnp.int32, (NW, L3_BINS), 1)
    col = jnp.sum(jnp.where(bins == b3, hi, 0), axis=1,
                  keepdims=True)
    wi = lax.broadcasted_iota(jnp.int32, (NW, NW), 0)
    wj = lax.broadcasted_iota(jnp.int32, (NW, NW), 1)
    col_b = jnp.broadcast_to(col.reshape(1, NW), (NW, NW))
    exi = jnp.sum(jnp.where(wj < wi, col_b, 0), axis=1).reshape(1, NW)
    exp_pad = jnp.concatenate(
        [jnp.zeros((1, 2), jnp.int32), exi,
         jnp.zeros((1, 128 - 2 - NW), jnp.int32)], axis=1)
    head = _lane_vec([(0, t), (1, rem)], default=0)
    lanes = lax.broadcasted_iota(jnp.int32, (1, 128), 1)
    out_ref[...] = jnp.where(lanes < 2, head, exp_pad)


def _pick(body, hist, *params):
    return pl.pallas_call(
        body, out_shape=jax.ShapeDtypeStruct((1, 128), jnp.int32),
    )(hist, *params)


def _final_body(nnz, ne, blk, scores_hbm, eidx_hbm, params_hbm,
                hard_hbm, accs_hbm, acch_hbm, accc_hbm,
                params_v, sc_b, e_b, hd_b, acc_s, acc_h, acc_c,
                sh_i, sh_f):
    chunk = nnz // NW
    nblk = chunk // blk
    grp = blk // L
    wid = _wid()
    base = wid * chunk
    iota = lax.iota(jnp.int32, L)

    pltpu.sync_copy(params_hbm.at[0], params_v)
    head = params_v[pl.ds(0, L)]
    t = head[0]
    rem = head[1]
    lane_pos = 2 + wid
    seg = params_v[pl.ds((lane_pos // L) * L, L)]
    base_eq = jnp.sum(jnp.where(iota == lane_pos % L, seg, 0))

    def za(i, _):
        z = jnp.zeros((L,), jnp.float32)
        acc_s[pl.ds(i * L, L)] = z
        acc_h[pl.ds(i * L, L)] = z
        acc_c[pl.ds(i * L, L)] = z
        return 0
    lax.fori_loop(0, ne // L, za, 0)

    def seg_add(acc_ref, cs, ends, e16):
        mcs = jnp.where(ends, cs, 0.0)
        cm = plsc.cummax(mcs)
        sh_f[...] = cm
        prev = plsc.load_gather(sh_f, [jnp.maximum(iota - 1, 0)])
        prev = jnp.where(iota == 0, 0.0, prev)
        plsc.addupdate_scatter(acc_ref, [e16], cs - prev, mask=ends)

    def blk_body(j, carry_eq):
        off = base + j * blk
        pltpu.sync_copy(scores_hbm.at[pl.ds(off, blk)], sc_b)
        pltpu.sync_copy(eidx_hbm.at[pl.ds(off, blk)], e_b)

        def grp_body(i, ce):
            s16 = sc_b[pl.ds(i * L, L)]
            key = plsc.bitcast(s16, jnp.int32)
            gt = key > t
            eq = key == t
            eqi = jnp.where(eq, 1, 0)
            csq = plsc.cumsum(eqi)
            rank = base_eq + ce + csq - eqi
            keep = jnp.logical_and(eq, rank < rem)
            hardv = jnp.where(jnp.logical_or(gt, keep), 1.0, 0.0)
            hd_b[pl.ds(i * L, L)] = hardv

            e16 = e_b[pl.ds(i * L, L)]
            sh_i[...] = e16
            e_next = plsc.load_gather(sh_i, [jnp.minimum(iota + 1, L - 1)])
            ends = jnp.logical_or(e16 != e_next, iota == L - 1)
            seg_add(acc_s, plsc.cumsum(s16), ends, e16)
            seg_add(acc_h, plsc.cumsum(hardv), ends, e16)
            seg_add(acc_c, (iota + 1).astype(jnp.float32), ends, e16)
            return ce + jnp.sum(eqi)
        carry_eq = lax.fori_loop(0, grp, grp_body, carry_eq)
        pltpu.sync_copy(hd_b, hard_hbm.at[pl.ds(off, blk)])
        return carry_eq
    lax.fori_loop(0, nblk, blk_body, jnp.int32(0))

    pltpu.sync_copy(acc_s, accs_hbm.at[wid])
    pltpu.sync_copy(acc_h, acch_hbm.at[wid])
    pltpu.sync_copy(acc_c, accc_hbm.at[wid])


def _final_pass(scores, e_idx, params3, ne, blk=2000):
    nnz = scores.shape[0]
    body = functools.partial(_final_body, nnz, ne, blk)
    f = pl.kernel(
        body,
        out_type=(jax.ShapeDtypeStruct((nnz,), jnp.float32),
                  jax.ShapeDtypeStruct((NW, ne), jnp.float32),
                  jax.ShapeDtypeStruct((NW, ne), jnp.float32),
                  jax.ShapeDtypeStruct((NW, ne), jnp.float32)),
        mesh=_MESH,
        compiler_params=pltpu.CompilerParams(needs_layout_passes=False,
                                             use_tc_tiling_on_sc=False),
        scratch_types=[
            pltpu.VMEM((128,), jnp.int32),
            pltpu.VMEM((blk,), jnp.float32),
            pltpu.VMEM((blk,), jnp.int32),
            pltpu.VMEM((blk,), jnp.float32),
            pltpu.VMEM((ne,), jnp.float32),
            pltpu.VMEM((ne,), jnp.float32),
            pltpu.VMEM((ne,), jnp.float32),
            pltpu.VMEM((L,), jnp.int32),
            pltpu.VMEM((L,), jnp.float32),
        ],
    )
    return f(scores, e_idx, params3)


def _edge_body(accs_ref, acch_ref, accc_ref, ep_ref, es_ref, eh_ref):
    s = jnp.sum(accs_ref[...], axis=0, keepdims=True)
    h = jnp.sum(acch_ref[...], axis=0, keepdims=True)
    c = jnp.sum(accc_ref[...], axis=0, keepdims=True)
    c1 = jnp.maximum(c, 1.0)
    ep_ref[...] = s / c1
    es_ref[...] = h / c1
    eh_ref[...] = jnp.where(h > 0.0, 1.0, 0.0)


def _edge_outputs(accs, acch, accc):
    ne = accs.shape[1]
    out = jax.ShapeDtypeStruct((1, ne), jnp.float32)
    ep, es, eh = pl.pallas_call(
        _edge_body, out_shape=(out, out, out),
    )(accs, acch, accc)
    return ep.reshape(ne), es.reshape(ne), eh.reshape(ne)


def kernel(x, V_idx, E_idx, edge_ids, W_node, edge_factors):
    nnz = V_idx.shape[0]
    ne = edge_ids.shape[0]
    k = max(1, int(0.5 * nnz))

    node_r = _node_proj(x, W_node)
    scores, hist1 = _scores_and_hist1(node_r, V_idx, E_idx, edge_ids,
                                      edge_factors)
    p1 = _pick(functools.partial(_pick1_body, k), hist1)
    hist2 = _hist_level(scores, p1, 2)
    p2 = _pick(_pick2_body, hist2, p1)
    hist3 = _hist_level(scores, p2, 3)
    p3 = _pick(_pick3_body, hist3, p2)
    hard, accs, acch, accc = _final_pass(scores, E_idx, p3, ne)
    edge_probs, edge_soft, edge_hard = _edge_outputs(accs, acch, accc)
    return (scores, hard, hard, edge_probs, edge_soft, edge_hard)
